# Initial kernel scaffold; baseline (speedup 1.0000x reference)
import jax
import jax.numpy as jnp
from jax.experimental import pallas as pl


def kernel(coord, atype, spin, force, virtual_scale_mask):
    raise NotImplementedError("write your pallas kernel here")


# speedup vs baseline: 1.3913x; 1.3913x over previous
import functools

import jax
import jax.numpy as jnp
from jax import lax
from jax.experimental import pallas as pl
from jax.experimental.pallas import tpu as pltpu
from jax.experimental.pallas import tpu_sc as plsc

_NUM_CORES = 2
_NUM_SUBCORES = 16
_NW = _NUM_CORES * _NUM_SUBCORES
_L = 16


def _sc_body(natom, ntypes, apw,
             coord_hbm, spin_hbm, atype_hbm, force_hbm, vsm_hbm,
             cs_hbm, as_hbm, fr_hbm, fm_hbm, mk_hbm,
             coord_v, spin_v, atype_v, freal_v, fmag_v,
             table_v, atspin_v, mask_v):
  wid = lax.axis_index("c") * _NUM_SUBCORES + lax.axis_index("s")
  wpf = natom // apw
  f = wid // wpf
  q = wid % wpf
  abase = f * natom + q * apw
  ebase = 3 * abase
  fr_off = f * (2 * 3 * natom) + q * (3 * apw)
  fm_off = fr_off + 3 * natom
  ar_off = f * (2 * natom) + q * apw
  av_off = ar_off + natom

  pltpu.sync_copy(vsm_hbm, table_v)
  pltpu.sync_copy(atype_hbm.at[pl.ds(abase, apw)], atype_v)
  pltpu.sync_copy(coord_hbm.at[pl.ds(ebase, 3 * apw)], coord_v)
  pltpu.sync_copy(spin_hbm.at[pl.ds(ebase, 3 * apw)], spin_v)
  pltpu.sync_copy(force_hbm.at[pl.ds(fr_off, 3 * apw)], freal_v)
  pltpu.sync_copy(force_hbm.at[pl.ds(fm_off, 3 * apw)], fmag_v)

  iota = lax.iota(jnp.int32, _L)
  offs = [(iota + (k * _L)) // 3 for k in range(3)]

  def body(g, carry):
    a0 = g * _L
    at = atype_v[pl.ds(a0, _L)]
    vm = plsc.load_gather(table_v, [at])
    atspin_v[pl.ds(a0, _L)] = at + ntypes
    mask_v[pl.ds(a0, _L)] = jnp.where(vm > 0.0, jnp.int32(1), jnp.int32(0))
    e0 = g * (3 * _L)
    for k in range(3):
      idx = a0 + offs[k]
      at3 = plsc.load_gather(atype_v, [idx])
      vm3 = plsc.load_gather(table_v, [at3])
      e = e0 + k * _L
      spin_v[pl.ds(e, _L)] = coord_v[pl.ds(e, _L)] + spin_v[pl.ds(e, _L)] * vm3
      fmag_v[pl.ds(e, _L)] = fmag_v[pl.ds(e, _L)] * vm3
    return carry

  lax.fori_loop(0, apw // _L, body, 0)

  pltpu.sync_copy(coord_v, cs_hbm.at[pl.ds(fr_off, 3 * apw)])
  pltpu.sync_copy(spin_v, cs_hbm.at[pl.ds(fm_off, 3 * apw)])
  pltpu.sync_copy(atype_v, as_hbm.at[pl.ds(ar_off, apw)])
  pltpu.sync_copy(atspin_v, as_hbm.at[pl.ds(av_off, apw)])
  pltpu.sync_copy(freal_v, fr_hbm.at[pl.ds(ebase, 3 * apw)])
  pltpu.sync_copy(fmag_v, fm_hbm.at[pl.ds(ebase, 3 * apw)])
  pltpu.sync_copy(mask_v, mk_hbm.at[pl.ds(abase, apw)])


def kernel(coord, atype, spin, force, virtual_scale_mask):
  nframes, natom = coord.shape[0], coord.shape[1]
  ntypes = virtual_scale_mask.shape[0]
  total_atoms = nframes * natom
  assert total_atoms % _NW == 0
  apw = total_atoms // _NW
  assert natom % apw == 0 and apw % _L == 0

  mesh = plsc.VectorSubcoreMesh(
      core_axis_name="c", subcore_axis_name="s",
      num_cores=_NUM_CORES, num_subcores=_NUM_SUBCORES)

  f32, i32 = jnp.float32, jnp.int32
  run = pl.kernel(
      functools.partial(_sc_body, natom, ntypes, apw),
      out_type=[
          jax.ShapeDtypeStruct((nframes * 2 * natom * 3,), f32),
          jax.ShapeDtypeStruct((nframes * 2 * natom,), i32),
          jax.ShapeDtypeStruct((nframes * natom * 3,), f32),
          jax.ShapeDtypeStruct((nframes * natom * 3,), f32),
          jax.ShapeDtypeStruct((nframes * natom,), i32),
      ],
      mesh=mesh,
      compiler_params=pltpu.CompilerParams(needs_layout_passes=False),
      scratch_types=[
          pltpu.VMEM((3 * apw,), f32),
          pltpu.VMEM((3 * apw,), f32),
          pltpu.VMEM((apw,), i32),
          pltpu.VMEM((3 * apw,), f32),
          pltpu.VMEM((3 * apw,), f32),
          pltpu.VMEM((ntypes,), f32),
          pltpu.VMEM((apw,), i32),
          pltpu.VMEM((apw,), i32),
      ],
  )

  cs, ats, fr, fm, mk = run(
      coord.reshape(-1), spin.reshape(-1), atype.reshape(-1),
      force.reshape(-1), virtual_scale_mask)

  coord_spin = cs.reshape(nframes, 2 * natom, 3)
  atype_spin = ats.reshape(nframes, 2 * natom)
  force_real = fr.reshape(nframes, natom, 3)
  force_mag = fm.reshape(nframes, natom, 3)
  atomic_mask = mk.reshape(nframes, natom, 1).astype(jnp.bool_)
  return coord_spin, atype_spin, force_real, force_mag, atomic_mask


# speedup vs baseline: 31.6861x; 22.7742x over previous
import functools

import jax
import jax.numpy as jnp
from jax import lax
from jax.experimental import pallas as pl
from jax.experimental.pallas import tpu as pltpu
from jax.experimental.pallas import tpu_sc as plsc

_NUM_CORES = 2
_NUM_SUBCORES = 16
_NW = _NUM_CORES * _NUM_SUBCORES
_L = 16


def _sc_body(nframes, natom, ntypes, cols,
             coord_hbm, spin_hbm, atype_hbm, force_hbm, vsm_hbm,
             cs_hbm, as_hbm, fr_hbm, fm_hbm, mk_hbm,
             coord_v, spin_v, atype_v, freal_v, fmag_v,
             table_v, atspin_v, mask_v):
  wid = lax.axis_index("c") * _NUM_SUBCORES + lax.axis_index("s")
  c0 = wid * cols

  pltpu.sync_copy(vsm_hbm, table_v)
  pltpu.sync_copy(atype_hbm.at[:, pl.ds(c0, cols)], atype_v)
  pltpu.sync_copy(coord_hbm.at[:, :, pl.ds(c0, cols)], coord_v)
  pltpu.sync_copy(spin_hbm.at[:, :, pl.ds(c0, cols)], spin_v)
  pltpu.sync_copy(force_hbm.at[:, :, pl.ds(c0, cols)], freal_v)
  pltpu.sync_copy(force_hbm.at[:, :, pl.ds(natom + c0, cols)], fmag_v)

  groups = cols // _L

  def body(g, carry):
    r = g // groups
    cc = (g % groups) * _L
    at = atype_v[r, pl.ds(cc, _L)]
    vm = plsc.load_gather(table_v, [at])
    atspin_v[r, pl.ds(cc, _L)] = at + ntypes
    mask_v[r, pl.ds(cc, _L)] = jnp.where(vm > 0.0, jnp.int32(1), jnp.int32(0))
    for p in range(3):
      spin_v[p, r, pl.ds(cc, _L)] = (
          coord_v[p, r, pl.ds(cc, _L)] + spin_v[p, r, pl.ds(cc, _L)] * vm)
      fmag_v[p, r, pl.ds(cc, _L)] = fmag_v[p, r, pl.ds(cc, _L)] * vm
    return carry

  lax.fori_loop(0, nframes * groups, body, 0)

  pltpu.sync_copy(coord_v, cs_hbm.at[:, :, pl.ds(c0, cols)])
  pltpu.sync_copy(spin_v, cs_hbm.at[:, :, pl.ds(natom + c0, cols)])
  pltpu.sync_copy(atype_v, as_hbm.at[:, pl.ds(c0, cols)])
  pltpu.sync_copy(atspin_v, as_hbm.at[:, pl.ds(natom + c0, cols)])
  pltpu.sync_copy(freal_v, fr_hbm.at[:, :, pl.ds(c0, cols)])
  pltpu.sync_copy(fmag_v, fm_hbm.at[:, :, pl.ds(c0, cols)])
  pltpu.sync_copy(mask_v, mk_hbm.at[:, pl.ds(c0, cols)])


def kernel(coord, atype, spin, force, virtual_scale_mask):
  nframes, natom = coord.shape[0], coord.shape[1]
  ntypes = virtual_scale_mask.shape[0]
  assert natom % _NW == 0
  cols = natom // _NW
  assert cols % _L == 0

  mesh = plsc.VectorSubcoreMesh(
      core_axis_name="c", subcore_axis_name="s",
      num_cores=_NUM_CORES, num_subcores=_NUM_SUBCORES)

  f32, i32 = jnp.float32, jnp.int32
  run = pl.kernel(
      functools.partial(_sc_body, nframes, natom, ntypes, cols),
      out_type=[
          jax.ShapeDtypeStruct((3, nframes, 2 * natom), f32),
          jax.ShapeDtypeStruct((nframes, 2 * natom), i32),
          jax.ShapeDtypeStruct((3, nframes, natom), f32),
          jax.ShapeDtypeStruct((3, nframes, natom), f32),
          jax.ShapeDtypeStruct((nframes, natom), i32),
      ],
      mesh=mesh,
      compiler_params=pltpu.CompilerParams(needs_layout_passes=False),
      scratch_types=[
          pltpu.VMEM((3, nframes, cols), f32),
          pltpu.VMEM((3, nframes, cols), f32),
          pltpu.VMEM((nframes, cols), i32),
          pltpu.VMEM((3, nframes, cols), f32),
          pltpu.VMEM((3, nframes, cols), f32),
          pltpu.VMEM((ntypes,), f32),
          pltpu.VMEM((nframes, cols), i32),
          pltpu.VMEM((nframes, cols), i32),
      ],
  )

  cs_t, ats, fr_t, fm_t, mk = run(
      jnp.transpose(coord, (2, 0, 1)), jnp.transpose(spin, (2, 0, 1)),
      atype, jnp.transpose(force, (2, 0, 1)), virtual_scale_mask)

  coord_spin = jnp.transpose(cs_t, (1, 2, 0))
  force_real = jnp.transpose(fr_t, (1, 2, 0))
  force_mag = jnp.transpose(fm_t, (1, 2, 0))
  atomic_mask = mk.reshape(nframes, natom, 1).astype(jnp.bool_)
  return coord_spin, ats, force_real, force_mag, atomic_mask


# speedup vs baseline: 34.6572x; 1.0938x over previous
import functools

import jax
import jax.numpy as jnp
from jax import lax
from jax.experimental import pallas as pl
from jax.experimental.pallas import tpu as pltpu
from jax.experimental.pallas import tpu_sc as plsc

_NUM_CORES = 2
_NUM_SUBCORES = 16
_NW = _NUM_CORES * _NUM_SUBCORES
_L = 16


def _sc_body(nframes, natom, ntypes, cols,
             coord_hbm, spin_hbm, atype_hbm, force_hbm, vsm_hbm,
             cs_hbm, as_hbm, fr_hbm, fm_hbm, mk_hbm,
             coord_v, spin_v, atype_v, freal_v, fmag_v,
             table_v, vmask_v, atspin_v, mask_v,
             sem_a, sem_c, sem_s, sem_fr, sem_fm, sem_o):
  wid = lax.axis_index("c") * _NUM_SUBCORES + lax.axis_index("s")
  c0 = wid * cols
  groups = cols // _L

  in_a = pltpu.async_copy(atype_hbm.at[:, pl.ds(c0, cols)], atype_v, sem_a)
  in_c = pltpu.async_copy(coord_hbm.at[:, :, pl.ds(c0, cols)], coord_v, sem_c)
  in_s = pltpu.async_copy(spin_hbm.at[:, :, pl.ds(c0, cols)], spin_v, sem_s)
  in_fm = pltpu.async_copy(
      force_hbm.at[:, :, pl.ds(natom + c0, cols)], fmag_v, sem_fm)
  in_fr = pltpu.async_copy(
      force_hbm.at[:, :, pl.ds(c0, cols)], freal_v, sem_fr)
  pltpu.sync_copy(vsm_hbm, table_v)

  in_a.wait()

  def body_atype(g, carry):
    r = g // groups
    cc = (g % groups) * _L
    at = atype_v[r, pl.ds(cc, _L)]
    vm = plsc.load_gather(table_v, [at])
    vmask_v[r, pl.ds(cc, _L)] = vm
    atspin_v[r, pl.ds(cc, _L)] = at + ntypes
    mask_v[r, pl.ds(cc, _L)] = jnp.where(vm > 0.0, jnp.int32(1), jnp.int32(0))
    return carry

  lax.fori_loop(0, nframes * groups, body_atype, 0)

  out_ar = pltpu.async_copy(atype_v, as_hbm.at[:, pl.ds(c0, cols)], sem_o)
  out_av = pltpu.async_copy(
      atspin_v, as_hbm.at[:, pl.ds(natom + c0, cols)], sem_o)
  out_mk = pltpu.async_copy(mask_v, mk_hbm.at[:, pl.ds(c0, cols)], sem_o)

  in_c.wait()
  out_cr = pltpu.async_copy(coord_v, cs_hbm.at[:, :, pl.ds(c0, cols)], sem_o)
  in_s.wait()

  def body_coord(g, carry):
    r = g // groups
    cc = (g % groups) * _L
    vm = vmask_v[r, pl.ds(cc, _L)]
    for p in range(3):
      spin_v[p, r, pl.ds(cc, _L)] = (
          coord_v[p, r, pl.ds(cc, _L)] + spin_v[p, r, pl.ds(cc, _L)] * vm)
    return carry

  lax.fori_loop(0, nframes * groups, body_coord, 0)
  out_cv = pltpu.async_copy(
      spin_v, cs_hbm.at[:, :, pl.ds(natom + c0, cols)], sem_o)

  in_fm.wait()

  def body_fmag(g, carry):
    r = g // groups
    cc = (g % groups) * _L
    vm = vmask_v[r, pl.ds(cc, _L)]
    for p in range(3):
      fmag_v[p, r, pl.ds(cc, _L)] = fmag_v[p, r, pl.ds(cc, _L)] * vm
    return carry

  lax.fori_loop(0, nframes * groups, body_fmag, 0)
  out_fm = pltpu.async_copy(fmag_v, fm_hbm.at[:, :, pl.ds(c0, cols)], sem_o)

  in_fr.wait()
  out_fr = pltpu.async_copy(freal_v, fr_hbm.at[:, :, pl.ds(c0, cols)], sem_o)

  out_ar.wait()
  out_av.wait()
  out_mk.wait()
  out_cr.wait()
  out_cv.wait()
  out_fm.wait()
  out_fr.wait()


def kernel(coord, atype, spin, force, virtual_scale_mask):
  nframes, natom = coord.shape[0], coord.shape[1]
  ntypes = virtual_scale_mask.shape[0]
  assert natom % _NW == 0
  cols = natom // _NW
  assert cols % _L == 0

  mesh = plsc.VectorSubcoreMesh(
      core_axis_name="c", subcore_axis_name="s",
      num_cores=_NUM_CORES, num_subcores=_NUM_SUBCORES)

  f32, i32 = jnp.float32, jnp.int32
  run = pl.kernel(
      functools.partial(_sc_body, nframes, natom, ntypes, cols),
      out_type=[
          jax.ShapeDtypeStruct((3, nframes, 2 * natom), f32),
          jax.ShapeDtypeStruct((nframes, 2 * natom), i32),
          jax.ShapeDtypeStruct((3, nframes, natom), f32),
          jax.ShapeDtypeStruct((3, nframes, natom), f32),
          jax.ShapeDtypeStruct((nframes, natom), i32),
      ],
      mesh=mesh,
      compiler_params=pltpu.CompilerParams(needs_layout_passes=False),
      scratch_types=[
          pltpu.VMEM((3, nframes, cols), f32),
          pltpu.VMEM((3, nframes, cols), f32),
          pltpu.VMEM((nframes, cols), i32),
          pltpu.VMEM((3, nframes, cols), f32),
          pltpu.VMEM((3, nframes, cols), f32),
          pltpu.VMEM((ntypes,), f32),
          pltpu.VMEM((nframes, cols), f32),
          pltpu.VMEM((nframes, cols), i32),
          pltpu.VMEM((nframes, cols), i32),
          pltpu.SemaphoreType.DMA,
          pltpu.SemaphoreType.DMA,
          pltpu.SemaphoreType.DMA,
          pltpu.SemaphoreType.DMA,
          pltpu.SemaphoreType.DMA,
          pltpu.SemaphoreType.DMA,
      ],
  )

  cs_t, ats, fr_t, fm_t, mk = run(
      jnp.transpose(coord, (2, 0, 1)), jnp.transpose(spin, (2, 0, 1)),
      atype, jnp.transpose(force, (2, 0, 1)), virtual_scale_mask)

  coord_spin = jnp.transpose(cs_t, (1, 2, 0))
  force_real = jnp.transpose(fr_t, (1, 2, 0))
  force_mag = jnp.transpose(fm_t, (1, 2, 0))
  atomic_mask = mk.reshape(nframes, natom, 1).astype(jnp.bool_)
  return coord_spin, ats, force_real, force_mag, atomic_mask


# speedup vs baseline: 35.1235x; 1.0135x over previous
import functools

import jax
import jax.numpy as jnp
from jax import lax
from jax.experimental import pallas as pl
from jax.experimental.pallas import tpu as pltpu
from jax.experimental.pallas import tpu_sc as plsc

_NUM_CORES = 2
_NUM_SUBCORES = 16
_NW = _NUM_CORES * _NUM_SUBCORES
_L = 16


def _sc_body(nframes, natom, ntypes, cols,
             coord_hbm, spin_hbm, atype_hbm, force_hbm, vsm_hbm,
             cs_hbm, as_hbm, fm_hbm, mk_hbm,
             coord_v, spin_v, atype_v, fmag_v,
             table_v, vmask_v, atspin_v, mask_v,
             sem_a, sem_c, sem_s, sem_fm, sem_o):
  wid = lax.axis_index("c") * _NUM_SUBCORES + lax.axis_index("s")
  c0 = wid * cols
  groups = cols // _L

  in_a = pltpu.async_copy(atype_hbm.at[:, pl.ds(c0, cols)], atype_v, sem_a)
  in_c = pltpu.async_copy(coord_hbm.at[:, :, pl.ds(c0, cols)], coord_v, sem_c)
  in_s = pltpu.async_copy(spin_hbm.at[:, :, pl.ds(c0, cols)], spin_v, sem_s)
  in_fm = pltpu.async_copy(
      force_hbm.at[:, :, pl.ds(natom + c0, cols)], fmag_v, sem_fm)
  pltpu.sync_copy(vsm_hbm, table_v)

  in_a.wait()

  def body_atype(g, carry):
    r = g // groups
    cc = (g % groups) * _L
    at = atype_v[r, pl.ds(cc, _L)]
    vm = plsc.load_gather(table_v, [at])
    vmask_v[r, pl.ds(cc, _L)] = vm
    atspin_v[r, pl.ds(cc, _L)] = at + ntypes
    mask_v[r, pl.ds(cc, _L)] = jnp.where(vm > 0.0, jnp.int32(1), jnp.int32(0))
    return carry

  lax.fori_loop(0, nframes * groups, body_atype, 0)

  out_ar = pltpu.async_copy(atype_v, as_hbm.at[:, pl.ds(c0, cols)], sem_o)
  out_av = pltpu.async_copy(
      atspin_v, as_hbm.at[:, pl.ds(natom + c0, cols)], sem_o)
  out_mk = pltpu.async_copy(mask_v, mk_hbm.at[:, pl.ds(c0, cols)], sem_o)

  in_c.wait()
  out_cr = pltpu.async_copy(coord_v, cs_hbm.at[:, :, pl.ds(c0, cols)], sem_o)
  in_s.wait()

  def body_coord(g, carry):
    r = g // groups
    cc = (g % groups) * _L
    vm = vmask_v[r, pl.ds(cc, _L)]
    for p in range(3):
      spin_v[p, r, pl.ds(cc, _L)] = (
          coord_v[p, r, pl.ds(cc, _L)] + spin_v[p, r, pl.ds(cc, _L)] * vm)
    return carry

  lax.fori_loop(0, nframes * groups, body_coord, 0)
  out_cv = pltpu.async_copy(
      spin_v, cs_hbm.at[:, :, pl.ds(natom + c0, cols)], sem_o)

  in_fm.wait()

  def body_fmag(g, carry):
    r = g // groups
    cc = (g % groups) * _L
    vm = vmask_v[r, pl.ds(cc, _L)]
    for p in range(3):
      fmag_v[p, r, pl.ds(cc, _L)] = fmag_v[p, r, pl.ds(cc, _L)] * vm
    return carry

  lax.fori_loop(0, nframes * groups, body_fmag, 0)
  out_fm = pltpu.async_copy(fmag_v, fm_hbm.at[:, :, pl.ds(c0, cols)], sem_o)

  out_ar.wait()
  out_av.wait()
  out_mk.wait()
  out_cr.wait()
  out_cv.wait()
  out_fm.wait()


def kernel(coord, atype, spin, force, virtual_scale_mask):
  nframes, natom = coord.shape[0], coord.shape[1]
  ntypes = virtual_scale_mask.shape[0]
  assert natom % _NW == 0
  cols = natom // _NW
  assert cols % _L == 0

  mesh = plsc.VectorSubcoreMesh(
      core_axis_name="c", subcore_axis_name="s",
      num_cores=_NUM_CORES, num_subcores=_NUM_SUBCORES)

  f32, i32 = jnp.float32, jnp.int32
  run = pl.kernel(
      functools.partial(_sc_body, nframes, natom, ntypes, cols),
      out_type=[
          jax.ShapeDtypeStruct((3, nframes, 2 * natom), f32),
          jax.ShapeDtypeStruct((nframes, 2 * natom), i32),
          jax.ShapeDtypeStruct((3, nframes, natom), f32),
          jax.ShapeDtypeStruct((nframes, natom), i32),
      ],
      mesh=mesh,
      compiler_params=pltpu.CompilerParams(needs_layout_passes=False),
      scratch_types=[
          pltpu.VMEM((3, nframes, cols), f32),
          pltpu.VMEM((3, nframes, cols), f32),
          pltpu.VMEM((nframes, cols), i32),
          pltpu.VMEM((3, nframes, cols), f32),
          pltpu.VMEM((ntypes,), f32),
          pltpu.VMEM((nframes, cols), f32),
          pltpu.VMEM((nframes, cols), i32),
          pltpu.VMEM((nframes, cols), i32),
          pltpu.SemaphoreType.DMA,
          pltpu.SemaphoreType.DMA,
          pltpu.SemaphoreType.DMA,
          pltpu.SemaphoreType.DMA,
          pltpu.SemaphoreType.DMA,
      ],
  )

  cs_t, ats, fm_t, mk = run(
      jnp.transpose(coord, (2, 0, 1)), jnp.transpose(spin, (2, 0, 1)),
      atype, jnp.transpose(force, (2, 0, 1)), virtual_scale_mask)

  coord_spin = jnp.transpose(cs_t, (1, 2, 0))
  force_real = force[:, :natom]
  force_mag = jnp.transpose(fm_t, (1, 2, 0))
  atomic_mask = mk.reshape(nframes, natom, 1).astype(jnp.bool_)
  return coord_spin, ats, force_real, force_mag, atomic_mask


# speedup vs baseline: 35.2062x; 1.0024x over previous
import functools

import jax
import jax.numpy as jnp
from jax import lax
from jax.experimental import pallas as pl
from jax.experimental.pallas import tpu as pltpu
from jax.experimental.pallas import tpu_sc as plsc

_NUM_CORES = 2
_NUM_SUBCORES = 16
_NW = _NUM_CORES * _NUM_SUBCORES
_L = 16


def _sc_body(nframes, natom, ntypes, cols,
             coord_hbm, spin_hbm, atype_hbm, force_hbm, vsm_hbm,
             cs_hbm, as_hbm, fm_hbm, mk_hbm,
             coord_v, spin_v, atype_v, fmag_v,
             table_v, vmask_v, atspin_v, mask_v,
             sem_a, sem_c, sem_s, sem_fm, sem_o):
  wid = lax.axis_index("c") * _NUM_SUBCORES + lax.axis_index("s")
  c0 = wid * cols
  groups = cols // _L

  in_a = pltpu.async_copy(atype_hbm.at[:, pl.ds(c0, cols)], atype_v, sem_a)
  in_c = pltpu.async_copy(coord_hbm.at[:, :, pl.ds(c0, cols)], coord_v, sem_c)
  in_s = pltpu.async_copy(spin_hbm.at[:, :, pl.ds(c0, cols)], spin_v, sem_s)
  in_fm = pltpu.async_copy(
      force_hbm.at[:, :, pl.ds(natom + c0, cols)], fmag_v, sem_fm)
  pltpu.sync_copy(vsm_hbm, table_v)

  in_a.wait()

  def body_atype(g, carry):
    r = g // groups
    cc = (g % groups) * _L
    at = atype_v[r, pl.ds(cc, _L)]
    vm = plsc.load_gather(table_v, [at])
    vmask_v[r, pl.ds(cc, _L)] = vm
    atspin_v[r, pl.ds(cc, _L)] = at + ntypes
    mask_v[r, pl.ds(cc, _L)] = jnp.where(vm > 0.0, jnp.int32(1), jnp.int32(0))
    return carry

  lax.fori_loop(0, nframes * groups, body_atype, 0)

  out_ar = pltpu.async_copy(atype_v, as_hbm.at[:, pl.ds(c0, cols)], sem_o)
  out_av = pltpu.async_copy(
      atspin_v, as_hbm.at[:, pl.ds(natom + c0, cols)], sem_o)
  out_mk = pltpu.async_copy(mask_v, mk_hbm.at[:, pl.ds(c0, cols)], sem_o)

  in_c.wait()
  out_cr = pltpu.async_copy(coord_v, cs_hbm.at[:, :, pl.ds(c0, cols)], sem_o)
  in_s.wait()

  def body_coord(g, carry):
    r = g // groups
    cc = (g % groups) * _L
    vm = vmask_v[r, pl.ds(cc, _L)]
    for p in range(3):
      spin_v[p, r, pl.ds(cc, _L)] = (
          coord_v[p, r, pl.ds(cc, _L)] + spin_v[p, r, pl.ds(cc, _L)] * vm)
    return carry

  lax.fori_loop(0, nframes * groups, body_coord, 0)
  out_cv = pltpu.async_copy(
      spin_v, cs_hbm.at[:, :, pl.ds(natom + c0, cols)], sem_o)

  in_fm.wait()

  def body_fmag(g, carry):
    r = g // groups
    cc = (g % groups) * _L
    vm = vmask_v[r, pl.ds(cc, _L)]
    for p in range(3):
      fmag_v[p, r, pl.ds(cc, _L)] = fmag_v[p, r, pl.ds(cc, _L)] * vm
    return carry

  lax.fori_loop(0, nframes * groups, body_fmag, 0)
  out_fm = pltpu.async_copy(fmag_v, fm_hbm.at[:, :, pl.ds(c0, cols)], sem_o)

  out_ar.wait()
  out_av.wait()
  out_mk.wait()
  out_cr.wait()
  out_cv.wait()
  out_fm.wait()


def kernel(coord, atype, spin, force, virtual_scale_mask):
  nframes, natom = coord.shape[0], coord.shape[1]
  ntypes = virtual_scale_mask.shape[0]
  assert natom % _NW == 0
  cols = natom // _NW
  assert cols % _L == 0

  mesh = plsc.VectorSubcoreMesh(
      core_axis_name="c", subcore_axis_name="s",
      num_cores=_NUM_CORES, num_subcores=_NUM_SUBCORES)

  f32, i32 = jnp.float32, jnp.int32
  run = pl.kernel(
      functools.partial(_sc_body, nframes, natom, ntypes, cols),
      out_type=[
          jax.ShapeDtypeStruct((3, nframes, 2 * natom), f32),
          jax.ShapeDtypeStruct((nframes, 2 * natom), i32),
          jax.ShapeDtypeStruct((3, nframes, natom), f32),
          jax.ShapeDtypeStruct((nframes, natom), i32),
      ],
      mesh=mesh,
      compiler_params=pltpu.CompilerParams(
          needs_layout_passes=False, skip_device_barrier=True),
      scratch_types=[
          pltpu.VMEM((3, nframes, cols), f32),
          pltpu.VMEM((3, nframes, cols), f32),
          pltpu.VMEM((nframes, cols), i32),
          pltpu.VMEM((3, nframes, cols), f32),
          pltpu.VMEM((ntypes,), f32),
          pltpu.VMEM((nframes, cols), f32),
          pltpu.VMEM((nframes, cols), i32),
          pltpu.VMEM((nframes, cols), i32),
          pltpu.SemaphoreType.DMA,
          pltpu.SemaphoreType.DMA,
          pltpu.SemaphoreType.DMA,
          pltpu.SemaphoreType.DMA,
          pltpu.SemaphoreType.DMA,
      ],
  )

  cs_t, ats, fm_t, mk = run(
      jnp.transpose(coord, (2, 0, 1)), jnp.transpose(spin, (2, 0, 1)),
      atype, jnp.transpose(force, (2, 0, 1)), virtual_scale_mask)

  coord_spin = jnp.transpose(cs_t, (1, 2, 0))
  force_real = force[:, :natom]
  force_mag = jnp.transpose(fm_t, (1, 2, 0))
  atomic_mask = mk.reshape(nframes, natom, 1).astype(jnp.bool_)
  return coord_spin, ats, force_real, force_mag, atomic_mask


# speedup vs baseline: 36.0634x; 1.0243x over previous
import functools

import jax
import jax.numpy as jnp
from jax import lax
from jax.experimental import pallas as pl
from jax.experimental.pallas import tpu as pltpu
from jax.experimental.pallas import tpu_sc as plsc

_NUM_CORES = 2
_NUM_SUBCORES = 16
_NW = _NUM_CORES * _NUM_SUBCORES
_L = 16


def _sc_body(nframes, natom, ntypes, cols,
             coord_hbm, spin_hbm, atype_hbm, force_hbm, vsm_hbm,
             cs_hbm, as_hbm, fm_hbm, mk_hbm,
             coord_v, spin_v, atype_v, fmag_v,
             table_v, vmask_v, atspin_v, mask_v,
             sem_a, sem_c, sem_s, sem_fm, sem_o):
  wid = lax.axis_index("c") * _NUM_SUBCORES + lax.axis_index("s")
  c0 = wid * cols
  groups = cols // _L

  in_a = pltpu.async_copy(atype_hbm.at[:, pl.ds(c0, cols)], atype_v, sem_a)
  in_c = pltpu.async_copy(coord_hbm.at[:, :, pl.ds(c0, cols)], coord_v, sem_c)
  in_s = pltpu.async_copy(spin_hbm.at[:, :, pl.ds(c0, cols)], spin_v, sem_s)
  in_fm = pltpu.async_copy(
      force_hbm.at[:, :, pl.ds(natom + c0, cols)], fmag_v, sem_fm)
  pltpu.sync_copy(vsm_hbm, table_v)

  in_a.wait()

  def body_atype(g, carry):
    r = g // groups
    cc = (g % groups) * _L
    at = atype_v[r, pl.ds(cc, _L)]
    vm = plsc.load_gather(table_v, [at])
    vmask_v[r, pl.ds(cc, _L)] = vm
    atspin_v[r, pl.ds(cc, _L)] = at + ntypes
    mask_v[r, pl.ds(cc, _L)] = jnp.where(vm > 0.0, jnp.int32(1), jnp.int32(0))
    return carry

  lax.fori_loop(0, nframes * groups, body_atype, 0)

  out_ar = pltpu.async_copy(atype_v, as_hbm.at[:, pl.ds(c0, cols)], sem_o)
  out_av = pltpu.async_copy(
      atspin_v, as_hbm.at[:, pl.ds(natom + c0, cols)], sem_o)
  out_mk = pltpu.async_copy(mask_v, mk_hbm.at[:, pl.ds(c0, cols)], sem_o)

  in_c.wait()
  out_cr = pltpu.async_copy(coord_v, cs_hbm.at[:, :, pl.ds(c0, cols)], sem_o)
  in_s.wait()
  in_fm.wait()

  def body_vec(g, carry):
    r = g // groups
    cc = (g % groups) * _L
    vm = vmask_v[r, pl.ds(cc, _L)]
    for p in range(3):
      spin_v[p, r, pl.ds(cc, _L)] = (
          coord_v[p, r, pl.ds(cc, _L)] + spin_v[p, r, pl.ds(cc, _L)] * vm)
      fmag_v[p, r, pl.ds(cc, _L)] = fmag_v[p, r, pl.ds(cc, _L)] * vm
    return carry

  lax.fori_loop(0, nframes * groups, body_vec, 0)
  out_cv = pltpu.async_copy(
      spin_v, cs_hbm.at[:, :, pl.ds(natom + c0, cols)], sem_o)
  out_fm = pltpu.async_copy(fmag_v, fm_hbm.at[:, :, pl.ds(c0, cols)], sem_o)

  out_ar.wait()
  out_av.wait()
  out_mk.wait()
  out_cr.wait()
  out_cv.wait()
  out_fm.wait()


def kernel(coord, atype, spin, force, virtual_scale_mask):
  nframes, natom = coord.shape[0], coord.shape[1]
  ntypes = virtual_scale_mask.shape[0]
  assert natom % _NW == 0
  cols = natom // _NW
  assert cols % _L == 0

  mesh = plsc.VectorSubcoreMesh(
      core_axis_name="c", subcore_axis_name="s",
      num_cores=_NUM_CORES, num_subcores=_NUM_SUBCORES)

  f32, i32 = jnp.float32, jnp.int32
  run = pl.kernel(
      functools.partial(_sc_body, nframes, natom, ntypes, cols),
      out_type=[
          jax.ShapeDtypeStruct((3, nframes, 2 * natom), f32),
          jax.ShapeDtypeStruct((nframes, 2 * natom), i32),
          jax.ShapeDtypeStruct((3, nframes, natom), f32),
          jax.ShapeDtypeStruct((nframes, natom), i32),
      ],
      mesh=mesh,
      compiler_params=pltpu.CompilerParams(needs_layout_passes=False),
      scratch_types=[
          pltpu.VMEM((3, nframes, cols), f32),
          pltpu.VMEM((3, nframes, cols), f32),
          pltpu.VMEM((nframes, cols), i32),
          pltpu.VMEM((3, nframes, cols), f32),
          pltpu.VMEM((ntypes,), f32),
          pltpu.VMEM((nframes, cols), f32),
          pltpu.VMEM((nframes, cols), i32),
          pltpu.VMEM((nframes, cols), i32),
          pltpu.SemaphoreType.DMA,
          pltpu.SemaphoreType.DMA,
          pltpu.SemaphoreType.DMA,
          pltpu.SemaphoreType.DMA,
          pltpu.SemaphoreType.DMA,
      ],
  )

  cs_t, ats, fm_t, mk = run(
      jnp.transpose(coord, (2, 0, 1)), jnp.transpose(spin, (2, 0, 1)),
      atype, jnp.transpose(force, (2, 0, 1)), virtual_scale_mask)

  coord_spin = jnp.transpose(cs_t, (1, 2, 0))
  force_real = force[:, :natom]
  force_mag = jnp.transpose(fm_t, (1, 2, 0))
  atomic_mask = mk.reshape(nframes, natom, 1).astype(jnp.bool_)
  return coord_spin, ats, force_real, force_mag, atomic_mask
